# Initial kernel scaffold; baseline (speedup 1.0000x reference)
import jax
import jax.numpy as jnp
from jax.experimental import pallas as pl


def kernel(q_color, q_oracle, q_emb_input, W_color, W_oracle, W_latent):
    raise NotImplementedError("write your pallas kernel here")


# speedup vs baseline: 2.5491x; 2.5491x over previous
import functools

import jax
import jax.numpy as jnp
from jax import lax
from jax.experimental import pallas as pl
from jax.experimental.pallas import tpu as pltpu
from jax.experimental.pallas import tpu_sc as plsc

B = 16384
N_COLORS = 32
DIM_COLOR = 16
OV = 34
L_OR = 20
DIM_ORACLE = 32
DIM_LATENT = 64


_NC = 2
_NS = 16
_NW = _NC * _NS
_BPW = B // _NW
_CH = 128
_NCH = _BPW // _CH


def _latent_body(idx_hbm, table_hbm, out_hbm, idx_v, rows_v, sem):
    wid = lax.axis_index("s") * _NC + lax.axis_index("c")
    base = wid * _BPW
    pltpu.sync_copy(idx_hbm.at[wid], idx_v)
    copies = [
        pltpu.async_copy(
            table_hbm.at[idx_v.at[j]],
            rows_v.at[pl.ds(j * _CH, _CH)],
            sem,
        )
        for j in range(_NCH)
    ]
    for c in copies:
        c.wait()
    pltpu.sync_copy(rows_v, out_hbm.at[pl.ds(base, _BPW)])


@functools.cache
def _latent_call():
    return functools.partial(
        pl.kernel,
        mesh=plsc.VectorSubcoreMesh(core_axis_name="c", subcore_axis_name="s"),
        out_type=jax.ShapeDtypeStruct((B, DIM_LATENT), jnp.float32),
        scratch_types=[
            pltpu.VMEM((_NCH, _CH), jnp.int32),
            pltpu.VMEM((_BPW, DIM_LATENT), jnp.float32),
            pltpu.SemaphoreType.DMA,
        ],
        compiler_params=pltpu.CompilerParams(use_tc_tiling_on_sc=False),
    )(_latent_body)


_BLK = 1024
_GRID = B // _BLK


def _co_body(qc_ref, qo_ref, wc_ref, wo_ref, out_ref):
    qc = qc_ref[0]
    qo = qo_ref[0]
    oh_c = (qc == lax.broadcasted_iota(jnp.int32, (_BLK, N_COLORS), 1)).astype(
        jnp.float32
    )
    color = jnp.dot(oh_c, wc_ref[:, :], preferred_element_type=jnp.float32)
    cnt = jnp.zeros((_BLK, OV), jnp.float32)
    for l in range(L_OR):
        cnt = cnt + (
            qo[:, l : l + 1] == lax.broadcasted_iota(jnp.int32, (_BLK, OV), 1)
        ).astype(jnp.float32)
    oracle = jnp.dot(cnt, wo_ref[:, :], preferred_element_type=jnp.float32) * (
        1.0 / L_OR
    )
    out_ref[0] = jnp.concatenate([color, oracle], axis=1)


def _make_co_call(interpret=False):
    return pl.pallas_call(
        _co_body,
        grid=(_GRID,),
        in_specs=[
            pl.BlockSpec((1, _BLK, 1), lambda i: (i, 0, 0)),
            pl.BlockSpec((1, _BLK, L_OR), lambda i: (i, 0, 0)),
            pl.BlockSpec((N_COLORS, DIM_COLOR), lambda i: (0, 0)),
            pl.BlockSpec((OV, DIM_ORACLE), lambda i: (0, 0)),
        ],
        out_specs=pl.BlockSpec((1, _BLK, DIM_COLOR + DIM_ORACLE), lambda i: (i, 0, 0)),
        out_shape=jax.ShapeDtypeStruct(
            (_GRID, _BLK, DIM_COLOR + DIM_ORACLE), jnp.float32
        ),
        interpret=interpret,
    )


_co_call = _make_co_call()


def kernel(q_color, q_oracle, q_emb_input, W_color, W_oracle, W_latent):
    qc3 = q_color.astype(jnp.int32).reshape(_GRID, _BLK, 1)
    qo3 = q_oracle.astype(jnp.int32).reshape(_GRID, _BLK, L_OR)
    co = _co_call(qc3, qo3, W_color, W_oracle).reshape(B, DIM_COLOR + DIM_ORACLE)
    idx = q_emb_input.astype(jnp.int32).reshape(_NW, _NCH, _CH)
    latent = _latent_call()(idx, W_latent)
    return jnp.concatenate([co, latent], axis=1)


# speedup vs baseline: 2.5748x; 1.0101x over previous
import functools

import jax
import jax.numpy as jnp
from jax import lax
from jax.experimental import pallas as pl
from jax.experimental.pallas import tpu as pltpu
from jax.experimental.pallas import tpu_sc as plsc

B = 16384
N_COLORS = 32
DIM_COLOR = 16
OV = 34
L_OR = 20
DIM_ORACLE = 32
DIM_LATENT = 64


_NC = 2
_NS = 16
_NW = _NC * _NS
_BPW = B // _NW
_CH = 128
_NCH = _BPW // _CH


def _latent_body(idx_hbm, table_hbm, out_hbm, idx_v, rows_v, sem):
    wid = lax.axis_index("s") * _NC + lax.axis_index("c")
    base = wid * _BPW
    pltpu.sync_copy(idx_hbm.at[wid], idx_v)
    copies = [
        pltpu.async_copy(
            table_hbm.at[idx_v.at[j]],
            rows_v.at[pl.ds(j * _CH, _CH)],
            sem,
        )
        for j in range(_NCH)
    ]
    for c in copies:
        c.wait()
    pltpu.sync_copy(rows_v, out_hbm.at[pl.ds(base, _BPW)])


@functools.cache
def _latent_call():
    return functools.partial(
        pl.kernel,
        mesh=plsc.VectorSubcoreMesh(core_axis_name="c", subcore_axis_name="s"),
        out_type=jax.ShapeDtypeStruct((B, DIM_LATENT), jnp.float32),
        scratch_types=[
            pltpu.VMEM((_NCH, _CH), jnp.int32),
            pltpu.VMEM((_BPW, DIM_LATENT), jnp.float32),
            pltpu.SemaphoreType.DMA,
        ],
        compiler_params=pltpu.CompilerParams(use_tc_tiling_on_sc=False),
    )(_latent_body)


_BLK = 1024
_GRID = B // _BLK


_DOUT = DIM_COLOR + DIM_ORACLE + DIM_LATENT


def _co_body(qc_ref, qo_ref, lat_ref, wc_ref, wo_ref, out_ref):
    qc = qc_ref[0]
    qo = qo_ref[0]
    oh_c = (qc == lax.broadcasted_iota(jnp.int32, (_BLK, N_COLORS), 1)).astype(
        jnp.float32
    )
    color = jnp.dot(oh_c, wc_ref[:, :], preferred_element_type=jnp.float32)
    cnt = jnp.zeros((_BLK, OV), jnp.float32)
    for l in range(L_OR):
        cnt = cnt + (
            qo[:, l : l + 1] == lax.broadcasted_iota(jnp.int32, (_BLK, OV), 1)
        ).astype(jnp.float32)
    oracle = jnp.dot(cnt, wo_ref[:, :], preferred_element_type=jnp.float32) * (
        1.0 / L_OR
    )
    out_ref[0] = jnp.concatenate([color, oracle, lat_ref[0]], axis=1)


def _make_co_call(interpret=False):
    return pl.pallas_call(
        _co_body,
        grid=(_GRID,),
        in_specs=[
            pl.BlockSpec((1, _BLK, 1), lambda i: (i, 0, 0)),
            pl.BlockSpec((1, _BLK, L_OR), lambda i: (i, 0, 0)),
            pl.BlockSpec((1, _BLK, DIM_LATENT), lambda i: (i, 0, 0)),
            pl.BlockSpec((N_COLORS, DIM_COLOR), lambda i: (0, 0)),
            pl.BlockSpec((OV, DIM_ORACLE), lambda i: (0, 0)),
        ],
        out_specs=pl.BlockSpec((1, _BLK, _DOUT), lambda i: (i, 0, 0)),
        out_shape=jax.ShapeDtypeStruct((_GRID, _BLK, _DOUT), jnp.float32),
        interpret=interpret,
    )


_co_call = _make_co_call()


def kernel(q_color, q_oracle, q_emb_input, W_color, W_oracle, W_latent):
    qc3 = q_color.astype(jnp.int32).reshape(_GRID, _BLK, 1)
    qo3 = q_oracle.astype(jnp.int32).reshape(_GRID, _BLK, L_OR)
    idx = q_emb_input.astype(jnp.int32).reshape(_NW, _NCH, _CH)
    latent = _latent_call()(idx, W_latent).reshape(_GRID, _BLK, DIM_LATENT)
    out = _co_call(qc3, qo3, latent, W_color, W_oracle)
    return out.reshape(B, _DOUT)
